# Initial kernel scaffold; baseline (speedup 1.0000x reference)
import jax
import jax.numpy as jnp
from jax.experimental import pallas as pl


def kernel(x, edge_index, batch, user_table, item_table, W1, b1, W2, b2, W_gcn, b_gcn, bn_gamma, bn_beta, bn_mean, bn_var, W_sage_l, W_sage_r, b_sage, W_cheb0, W_cheb1, b_cheb, W_gat, a_src, a_dst, b_gat, W_pred, b_pred):
    raise NotImplementedError("write your pallas kernel here")


# speedup vs baseline: 7.2106x; 7.2106x over previous
import functools

import jax
import jax.numpy as jnp
from jax import lax
from jax.experimental import pallas as pl
from jax.experimental.pallas import tpu as pltpu
from jax.experimental.pallas import tpu_sc as plsc

N = 10000
E = 160000
D = 128
EMB = 64
NC, NS, L = 2, 16, 16
NW = NC * NS
WRB = 624
TAIL = N - NS * WRB
ECH = 128
NECH = E // ECH
EIT = -(-NECH // NW)
VCH = 80
NVCH = N // VCH
VIT = -(-NVCH // NW)
ND = N + L

_MESH = plsc.VectorSubcoreMesh(
    core_axis_name="c", subcore_axis_name="s", num_cores=NC, num_subcores=NS)

F32 = jnp.float32


def _zero_core_rows(sh, zin, s):
    rs = s * WRB
    pltpu.sync_copy(zin, sh.at[pl.ds(rs, WRB), :])

    @pl.when(s == NS - 1)
    def _():
        pltpu.sync_copy(zin.at[pl.ds(0, TAIL), :],
                        sh.at[pl.ds(NS * WRB, TAIL), :])


def _writeout_core_rows(sh, out, c, s):
    rs = s * WRB
    pltpu.sync_copy(sh.at[pl.ds(rs, WRB), :], out.at[c, pl.ds(rs, WRB), :])

    @pl.when(s == NS - 1)
    def _():
        pltpu.sync_copy(sh.at[pl.ds(NS * WRB, TAIL), :],
                        out.at[c, pl.ds(NS * WRB, TAIL), :])


@functools.partial(
    pl.kernel,
    out_type=(
        jax.ShapeDtypeStruct((N, 2 * EMB), F32),
        jax.ShapeDtypeStruct((N, 2 * EMB), F32),
        jax.ShapeDtypeStruct((NC, N, D), F32),
    ),
    mesh=_MESH,
    scratch_types=[
        pltpu.VMEM((VCH,), jnp.int32),
        pltpu.VMEM((VCH,), jnp.int32),
        pltpu.VMEM((VCH, 2 * EMB), F32),
        pltpu.VMEM((VCH, 2 * EMB), F32),
        pltpu.VMEM((ECH,), jnp.int32),
        pltpu.VMEM((ECH, D), F32),
        pltpu.VMEM_SHARED((ND, D), F32),
        pltpu.SemaphoreType.DMA,
    ],
)
def _sc_embed_deg(user_t, item_t, u_idx, i_idx, dst, ones_in, zin,
                  u_out, i_out, deg_out,
                  uix, iix, urows, irows, dix, ones_v, deg_sh, sem):
    c = lax.axis_index("c")
    s = lax.axis_index("s")
    wid = s * NC + c

    pltpu.sync_copy(ones_in, ones_v)
    _zero_core_rows(deg_sh, zin, s)
    plsc.subcore_barrier()

    def deg_it(it, _):
        cid = it * NW + wid

        @pl.when(cid < NECH)
        def _():
            base = cid * ECH
            pltpu.sync_copy(dst.at[pl.ds(base, ECH)], dix)
            pltpu.sync_copy(ones_v, deg_sh.at[dix], add=True)
        return 0
    lax.fori_loop(0, EIT, deg_it, 0)
    plsc.subcore_barrier()
    _writeout_core_rows(deg_sh, deg_out, c, s)

    def emb_it(it, _):
        cid = it * NW + wid

        @pl.when(cid < NVCH)
        def _():
            base = cid * VCH
            pltpu.sync_copy(u_idx.at[pl.ds(base, VCH)], uix)
            pltpu.sync_copy(i_idx.at[pl.ds(base, VCH)], iix)
            cu = pltpu.async_copy(user_t.at[uix], urows, sem)
            ci = pltpu.async_copy(item_t.at[iix], irows, sem)
            cu.wait()
            ci.wait()
            pltpu.sync_copy(urows, u_out.at[pl.ds(base, VCH), :])
            pltpu.sync_copy(irows, i_out.at[pl.ds(base, VCH), :])
        return 0
    lax.fori_loop(0, VIT, emb_it, 0)


@functools.partial(
    pl.kernel,
    out_type=jax.ShapeDtypeStruct((NC, N, D), F32),
    mesh=_MESH,
    scratch_types=[
        pltpu.VMEM((ECH,), jnp.int32),
        pltpu.VMEM((ECH,), jnp.int32),
        pltpu.VMEM((ECH, D), F32),
        pltpu.VMEM_SHARED((ND, D), F32),
        pltpu.SemaphoreType.DMA,
    ],
)
def _sc_edge_sum(g, src, dst, zin, acc_out, six, dix, rows, acc_sh, sem):
    c = lax.axis_index("c")
    s = lax.axis_index("s")
    wid = s * NC + c

    _zero_core_rows(acc_sh, zin, s)
    plsc.subcore_barrier()

    def edge_it(it, _):
        cid = it * NW + wid

        @pl.when(cid < NECH)
        def _():
            base = cid * ECH
            pltpu.sync_copy(src.at[pl.ds(base, ECH)], six)
            pltpu.sync_copy(dst.at[pl.ds(base, ECH)], dix)
            pltpu.async_copy(g.at[six], rows, sem).wait()
            pltpu.sync_copy(rows, acc_sh.at[dix], add=True)
        return 0
    lax.fori_loop(0, EIT, edge_it, 0)
    plsc.subcore_barrier()
    _writeout_core_rows(acc_sh, acc_out, c, s)


@functools.partial(
    pl.kernel,
    out_type=jax.ShapeDtypeStruct((E, D), F32),
    mesh=_MESH,
    scratch_types=[
        pltpu.VMEM((ECH,), jnp.int32),
        pltpu.VMEM((ECH,), jnp.int32),
        pltpu.VMEM((ECH, D), F32),
        pltpu.VMEM((ECH, D), F32),
        pltpu.SemaphoreType.DMA,
    ],
)
def _sc_expand(meta_s, meta_d, src, dst, comb_out,
               six, dix, rows_s, rows_d, sem):
    c = lax.axis_index("c")
    s = lax.axis_index("s")
    wid = s * NC + c

    def edge_it(it, _):
        cid = it * NW + wid

        @pl.when(cid < NECH)
        def _():
            base = cid * ECH
            pltpu.sync_copy(src.at[pl.ds(base, ECH)], six)
            pltpu.sync_copy(dst.at[pl.ds(base, ECH)], dix)
            cs = pltpu.async_copy(meta_s.at[six], rows_s, sem)
            cd = pltpu.async_copy(meta_d.at[dix], rows_d, sem)
            cs.wait()
            cd.wait()

            def row_add(e, _):
                rows_s[e, pl.ds(0, L)] = (rows_s[e, pl.ds(0, L)]
                                          + rows_d[e, pl.ds(0, L)])
                return 0
            lax.fori_loop(0, ECH, row_add, 0)
            pltpu.sync_copy(rows_s, comb_out.at[pl.ds(base, ECH), :])
        return 0
    lax.fori_loop(0, EIT, edge_it, 0)


R = 1000
G = N // R


def _row_spec(cols):
    return pl.BlockSpec((R, cols), lambda i: (i, 0))


def _deg_spec():
    return pl.BlockSpec((NC, R, D), lambda i: (0, i, 0))


def _full_spec(r, cols):
    return pl.BlockSpec((r, cols), lambda i: (0, 0))


def _indeg(dref):
    return dref[0, :, 0] + dref[1, :, 0]


def _tca_body(u, i, pu, pi, d, W1u, W1i, b1, W2, b2, Wg, hg_o, g_o):
    lane = lax.broadcasted_iota(jnp.int32, (R, 2 * EMB), 1)
    low = (lane < EMB).astype(F32)
    mu = low * (1.0 - pu[:]) + (1.0 - low) * pu[:]
    mi = low * (1.0 - pi[:]) + (1.0 - low) * pi[:]
    h1 = jnp.maximum(
        jnp.dot(u[:] * mu, W1u[:], preferred_element_type=F32)
        + jnp.dot(i[:] * mi, W1i[:], preferred_element_type=F32)
        + b1[:], 0.0)
    h2 = jnp.maximum(jnp.dot(h1, W2[:], preferred_element_type=F32) + b2[:], 0.0)
    hg = jnp.dot(h2, Wg[:], preferred_element_type=F32)
    dinv = lax.rsqrt(_indeg(d) + 1.0)
    hg_o[:] = hg
    g_o[:] = hg * dinv[:, None]


def _tc_mlp(u_pack, i_pack, pu, pi, deg, W1u, W1i, b1, W2, b2, Wg):
    return pl.pallas_call(
        _tca_body,
        grid=(G,),
        in_specs=[
            _row_spec(2 * EMB), _row_spec(2 * EMB),
            _row_spec(1), _row_spec(1), _deg_spec(),
            _full_spec(2 * EMB, 1024), _full_spec(2 * EMB, 1024),
            _full_spec(1, 1024),
            _full_spec(1024, 512), _full_spec(1, 512),
            _full_spec(512, D),
        ],
        out_specs=[_row_spec(D), _row_spec(D)],
        out_shape=[
            jax.ShapeDtypeStruct((N, D), F32),
            jax.ShapeDtypeStruct((N, D), F32),
        ],
    )(u_pack, i_pack, pu, pi, deg, W1u, W1i, b1, W2, b2, Wg)


def _tcb_body(a0, a1, hg, d, bg, gam, bet, mu, var, o_ref):
    dinv = lax.rsqrt(_indeg(d) + 1.0)
    o = (a0[:] + a1[:]) * dinv[:, None] + hg[:] * (dinv * dinv)[:, None] + bg[:]
    o = gam[:] * (o - mu[:]) / jnp.sqrt(var[:] + 1e-5) + bet[:]
    o_ref[:] = jnp.maximum(o, 0.0)


def _tc_gcn_bn(a0, a1, hg, deg, bg, gam, bet, mu, var):
    return pl.pallas_call(
        _tcb_body,
        grid=(G,),
        in_specs=[
            _row_spec(D), _row_spec(D), _row_spec(D), _deg_spec(),
            _full_spec(1, D), _full_spec(1, D), _full_spec(1, D),
            _full_spec(1, D), _full_spec(1, D),
        ],
        out_specs=[_row_spec(D)],
        out_shape=[jax.ShapeDtypeStruct((N, D), F32)],
    )(a0, a1, hg, deg, bg, gam, bet, mu, var)[0]


def _tcc_body(a0, a1, hbn, d, Wl, Wr, bs, h3_o, gc_o):
    indeg = _indeg(d)
    aggr = (a0[:] + a1[:]) / jnp.maximum(indeg, 1.0)[:, None]
    h3 = jnp.maximum(
        jnp.dot(aggr, Wl[:], preferred_element_type=F32)
        + jnp.dot(hbn[:], Wr[:], preferred_element_type=F32) + bs[:], 0.0)
    dinv_c = jnp.where(indeg > 0, lax.rsqrt(jnp.maximum(indeg, 1.0)), 0.0)
    h3_o[:] = h3
    gc_o[:] = h3 * dinv_c[:, None]


def _tc_sage(a0, a1, hbn, deg, Wl, Wr, bs):
    return pl.pallas_call(
        _tcc_body,
        grid=(G,),
        in_specs=[
            _row_spec(D), _row_spec(D), _row_spec(D), _deg_spec(),
            _full_spec(D, D), _full_spec(D, D), _full_spec(1, D),
        ],
        out_specs=[_row_spec(D), _row_spec(D)],
        out_shape=[
            jax.ShapeDtypeStruct((N, D), F32),
            jax.ShapeDtypeStruct((N, D), F32),
        ],
    )(a0, a1, hbn, deg, Wl, Wr, bs)


def _tcd_body(t0, t1, h3, d, W0, W1c, bc, Wgat, avs, avd,
              hg4_o, g1_o, g2_o, ms_o, md_o, qm_o):
    indeg = _indeg(d)
    dinv_c = jnp.where(indeg > 0, lax.rsqrt(jnp.maximum(indeg, 1.0)), 0.0)
    tx1 = -(t0[:] + t1[:]) * dinv_c[:, None]
    h4 = jnp.maximum(
        jnp.dot(h3[:], W0[:], preferred_element_type=F32)
        + jnp.dot(tx1, W1c[:], preferred_element_type=F32) + bc[:], 0.0)
    hg4 = jnp.dot(h4, Wgat[:], preferred_element_type=F32)
    asv = jnp.sum(hg4 * avs[:], axis=-1)
    adv = jnp.sum(hg4 * avd[:], axis=-1)
    q1 = jnp.exp(asv)
    q2 = jnp.exp(0.2 * asv)
    lane = lax.broadcasted_iota(jnp.int32, (R, D), 1)
    l0 = (lane == 0).astype(F32)
    l1 = (lane == 1).astype(F32)
    hg4_o[:] = hg4
    g1_o[:] = hg4 * q1[:, None]
    g2_o[:] = hg4 * q2[:, None]
    ms_o[:] = asv[:, None] * l0
    md_o[:] = adv[:, None] * l1
    qm_o[:] = q1[:, None] * l0 + q2[:, None] * l1


def _tc_cheb_gatprep(t0, t1, h3, deg, W0, W1c, bc, Wgat, avs, avd):
    return pl.pallas_call(
        _tcd_body,
        grid=(G,),
        in_specs=[
            _row_spec(D), _row_spec(D), _row_spec(D), _deg_spec(),
            _full_spec(D, D), _full_spec(D, D), _full_spec(1, D),
            _full_spec(D, D), _full_spec(1, D), _full_spec(1, D),
        ],
        out_specs=[_row_spec(D), _row_spec(D), _row_spec(D),
                   _row_spec(D), _row_spec(D), _row_spec(D)],
        out_shape=[jax.ShapeDtypeStruct((N, D), F32)] * 6,
    )(t0, t1, h3, deg, W0, W1c, bc, Wgat, avs, avd)


EB = 4000
EG = E // EB


def _edge_spec(cols):
    return pl.BlockSpec((EB, cols), lambda i: (i, 0))


def _tcf_body(comb, dstb, dixa_o, dixb_o):
    t = comb[:, 0:1] + comb[:, 1:2]
    keep = t >= 0.0
    d = dstb[:]
    dixa_o[:] = jnp.where(keep, d, N)
    dixb_o[:] = jnp.where(keep, N, d)


def _tc_edge(comb, dst2d):
    return pl.pallas_call(
        _tcf_body,
        grid=(EG,),
        in_specs=[_edge_spec(D), _edge_spec(1)],
        out_specs=[_edge_spec(1), _edge_spec(1)],
        out_shape=[
            jax.ShapeDtypeStruct((E, 1), jnp.int32),
            jax.ShapeDtypeStruct((E, 1), jnp.int32),
        ],
    )(comb, dst2d)


def _tce_body(na0, na1, nb0, nb1, da0, da1, db0, db1, hg4, ms, md,
              bgat, Wp, bp, o_ref):
    a = ms[:, 0]
    b = md[:, 1]
    qd1 = jnp.exp(b)
    qd2 = jnp.exp(0.2 * b)
    t = a + b
    wself = jnp.exp(jnp.maximum(t, 0.2 * t))
    num = (qd1[:, None] * (na0[:] + na1[:])
           + qd2[:, None] * (nb0[:] + nb1[:])
           + wself[:, None] * hg4[:])
    den = (qd1 * (da0[:, 0] + da1[:, 0])
           + qd2 * (db0[:, 1] + db1[:, 1]) + wself)
    o = num / jnp.maximum(den, 1e-16)[:, None]
    h5 = o + bgat[:]
    h5 = jnp.where(h5 > 0, h5, jnp.exp(jnp.minimum(h5, 0.0)) - 1.0)
    o_ref[:] = jnp.dot(h5, Wp[:], preferred_element_type=F32) + bp[:]


def _tc_gat_pred(na0, na1, nb0, nb1, da0, da1, db0, db1, hg4, ms, md,
                 bgat, Wp, bp):
    return pl.pallas_call(
        _tce_body,
        grid=(G,),
        in_specs=[
            _row_spec(D), _row_spec(D), _row_spec(D), _row_spec(D),
            _row_spec(D), _row_spec(D), _row_spec(D), _row_spec(D),
            _row_spec(D), _row_spec(D), _row_spec(D),
            _full_spec(1, D), _full_spec(D, 1), _full_spec(1, 1),
        ],
        out_specs=[_row_spec(1)],
        out_shape=[jax.ShapeDtypeStruct((N, 1), F32)],
    )(na0, na1, nb0, nb1, da0, da1, db0, db1, hg4, ms, md, bgat, Wp, bp)[0]


def kernel(x, edge_index, batch, user_table, item_table, W1, b1, W2, b2,
           W_gcn, b_gcn, bn_gamma, bn_beta, bn_mean, bn_var, W_sage_l,
           W_sage_r, b_sage, W_cheb0, W_cheb1, b_cheb, W_gat, a_src, a_dst,
           b_gat, W_pred, b_pred):
    u_idx = x[:, 0].astype(jnp.int32)
    i_idx = x[:, 1].astype(jnp.int32)
    src = edge_index[0].astype(jnp.int32)
    dst = edge_index[1].astype(jnp.int32)
    user_p = user_table.reshape(-1, 2 * EMB)
    item_p = item_table.reshape(-1, 2 * EMB)
    pu = (u_idx & 1).astype(F32).reshape(N, 1)
    pi = (i_idx & 1).astype(F32).reshape(N, 1)
    W1u2 = jnp.concatenate([W1[0:EMB], W1[0:EMB]], axis=0)
    W1i2 = jnp.concatenate([W1[EMB:2 * EMB], W1[EMB:2 * EMB]], axis=0)
    ones_in = jnp.concatenate(
        [jnp.ones((ECH, 1), F32), jnp.zeros((ECH, D - 1), F32)], axis=1)
    zin = jnp.zeros((WRB, D), F32)
    b1r = b1.reshape(1, -1)
    b2r = b2.reshape(1, -1)
    bgr = b_gcn.reshape(1, -1)
    gam = bn_gamma.reshape(1, -1)
    bet = bn_beta.reshape(1, -1)
    mu = bn_mean.reshape(1, -1)
    var = bn_var.reshape(1, -1)
    bsr = b_sage.reshape(1, -1)
    bcr = b_cheb.reshape(1, -1)
    bga = b_gat.reshape(1, -1)

    u_pack, i_pack, deg = _sc_embed_deg(user_p, item_p, u_idx >> 1,
                                        i_idx >> 1, dst, ones_in, zin)
    hg, g = _tc_mlp(u_pack, i_pack, pu, pi, deg, W1u2, W1i2, b1r, W2, b2r,
                    W_gcn)

    acc = _sc_edge_sum(g, src, dst, zin)
    hbn = _tc_gcn_bn(acc[0], acc[1], hg, deg, bgr, gam, bet, mu, var)

    aggr = _sc_edge_sum(hbn, src, dst, zin)
    h3, gc = _tc_sage(aggr[0], aggr[1], hbn, deg, W_sage_l, W_sage_r, bsr)

    tx = _sc_edge_sum(gc, src, dst, zin)
    hg4, g1, g2, ms, md, qm = _tc_cheb_gatprep(
        tx[0], tx[1], h3, deg, W_cheb0, W_cheb1, bcr, W_gat, a_src, a_dst)

    comb = _sc_expand(ms, md, src, dst)
    dixa, dixb = _tc_edge(comb, dst.reshape(E, 1))
    dixa = dixa.reshape(-1)
    dixb = dixb.reshape(-1)
    numa = _sc_edge_sum(g1, src, dixa, zin)
    numa, g2, dixb2 = lax.optimization_barrier((numa, g2, dixb))
    numb = _sc_edge_sum(g2, src, dixb2, zin)
    numb, qm1, dixa2 = lax.optimization_barrier((numb, qm, dixa))
    dena = _sc_edge_sum(qm1, src, dixa2, zin)
    dena, qm2, dixb3 = lax.optimization_barrier((dena, qm, dixb))
    denb = _sc_edge_sum(qm2, src, dixb3, zin)

    out = _tc_gat_pred(numa[0], numa[1], numb[0], numb[1],
                       dena[0], dena[1], denb[0], denb[1],
                       hg4, ms, md, bga, W_pred, b_pred.reshape(1, 1))
    return out.reshape(-1)


# speedup vs baseline: 7.4442x; 1.0324x over previous
import functools

import jax
import jax.numpy as jnp
from jax import lax
from jax.experimental import pallas as pl
from jax.experimental.pallas import tpu as pltpu
from jax.experimental.pallas import tpu_sc as plsc

N = 10000
E = 160000
D = 128
EMB = 64
NC, NS, L = 2, 16, 16
NW = NC * NS
WRB = 624
TAIL = N - NS * WRB
ECH = 128
NECH = E // ECH
EIT = -(-NECH // NW)
VCH = 80
NVCH = N // VCH
VIT = -(-NVCH // NW)
ND = N + L

_MESH = plsc.VectorSubcoreMesh(
    core_axis_name="c", subcore_axis_name="s", num_cores=NC, num_subcores=NS)

F32 = jnp.float32


def _zero_core_rows(sh, zin, s):
    rs = s * WRB
    pltpu.sync_copy(zin, sh.at[pl.ds(rs, WRB), :])

    @pl.when(s == NS - 1)
    def _():
        pltpu.sync_copy(zin.at[pl.ds(0, TAIL), :],
                        sh.at[pl.ds(NS * WRB, TAIL), :])


def _writeout_core_rows(sh, out, c, s):
    rs = s * WRB
    pltpu.sync_copy(sh.at[pl.ds(rs, WRB), :], out.at[c, pl.ds(rs, WRB), :])

    @pl.when(s == NS - 1)
    def _():
        pltpu.sync_copy(sh.at[pl.ds(NS * WRB, TAIL), :],
                        out.at[c, pl.ds(NS * WRB, TAIL), :])


@functools.partial(
    pl.kernel,
    out_type=(
        jax.ShapeDtypeStruct((N, 2 * EMB), F32),
        jax.ShapeDtypeStruct((N, 2 * EMB), F32),
        jax.ShapeDtypeStruct((NC, N, D), F32),
    ),
    mesh=_MESH,
    scratch_types=[
        pltpu.VMEM((VCH,), jnp.int32),
        pltpu.VMEM((VCH,), jnp.int32),
        pltpu.VMEM((VCH, 2 * EMB), F32),
        pltpu.VMEM((VCH, 2 * EMB), F32),
        pltpu.VMEM((ECH,), jnp.int32),
        pltpu.VMEM((ECH, D), F32),
        pltpu.VMEM_SHARED((ND, D), F32),
        pltpu.SemaphoreType.DMA,
    ],
)
def _sc_embed_deg(user_t, item_t, u_idx, i_idx, dst, ones_in, zin,
                  u_out, i_out, deg_out,
                  uix, iix, urows, irows, dix, ones_v, deg_sh, sem):
    c = lax.axis_index("c")
    s = lax.axis_index("s")
    wid = s * NC + c

    pltpu.sync_copy(ones_in, ones_v)
    _zero_core_rows(deg_sh, zin, s)
    plsc.subcore_barrier()

    def deg_it(it, _):
        cid = it * NW + wid

        @pl.when(cid < NECH)
        def _():
            base = cid * ECH
            pltpu.sync_copy(dst.at[pl.ds(base, ECH)], dix)
            pltpu.sync_copy(ones_v, deg_sh.at[dix], add=True)
        return 0
    lax.fori_loop(0, EIT, deg_it, 0)
    plsc.subcore_barrier()
    _writeout_core_rows(deg_sh, deg_out, c, s)

    def emb_it(it, _):
        cid = it * NW + wid

        @pl.when(cid < NVCH)
        def _():
            base = cid * VCH
            pltpu.sync_copy(u_idx.at[pl.ds(base, VCH)], uix)
            pltpu.sync_copy(i_idx.at[pl.ds(base, VCH)], iix)
            cu = pltpu.async_copy(user_t.at[uix], urows, sem)
            ci = pltpu.async_copy(item_t.at[iix], irows, sem)
            cu.wait()
            ci.wait()
            pltpu.sync_copy(urows, u_out.at[pl.ds(base, VCH), :])
            pltpu.sync_copy(irows, i_out.at[pl.ds(base, VCH), :])
        return 0
    lax.fori_loop(0, VIT, emb_it, 0)


@functools.partial(
    pl.kernel,
    out_type=jax.ShapeDtypeStruct((NC, N, D), F32),
    mesh=_MESH,
    scratch_types=[
        pltpu.VMEM((ECH,), jnp.int32),
        pltpu.VMEM((ECH,), jnp.int32),
        pltpu.VMEM((ECH, D), F32),
        pltpu.VMEM_SHARED((ND, D), F32),
        pltpu.SemaphoreType.DMA,
    ],
)
def _sc_edge_sum(g, src, dst, zin, acc_out, six, dix, rows, acc_sh, sem):
    c = lax.axis_index("c")
    s = lax.axis_index("s")
    wid = s * NC + c

    _zero_core_rows(acc_sh, zin, s)
    plsc.subcore_barrier()

    def edge_it(it, _):
        cid = it * NW + wid

        @pl.when(cid < NECH)
        def _():
            base = cid * ECH
            pltpu.sync_copy(src.at[pl.ds(base, ECH)], six)
            pltpu.sync_copy(dst.at[pl.ds(base, ECH)], dix)
            pltpu.async_copy(g.at[six], rows, sem).wait()
            pltpu.sync_copy(rows, acc_sh.at[dix], add=True)
        return 0
    lax.fori_loop(0, EIT, edge_it, 0)
    plsc.subcore_barrier()
    _writeout_core_rows(acc_sh, acc_out, c, s)


@functools.partial(
    pl.kernel,
    out_type=jax.ShapeDtypeStruct((E, D), F32),
    mesh=_MESH,
    scratch_types=[
        pltpu.VMEM((ECH,), jnp.int32),
        pltpu.VMEM((ECH,), jnp.int32),
        pltpu.VMEM((ECH, D), F32),
        pltpu.VMEM((ECH, D), F32),
        pltpu.SemaphoreType.DMA,
    ],
)
def _sc_expand(meta_s, meta_d, src, dst, comb_out,
               six, dix, rows_s, rows_d, sem):
    c = lax.axis_index("c")
    s = lax.axis_index("s")
    wid = s * NC + c

    def edge_it(it, _):
        cid = it * NW + wid

        @pl.when(cid < NECH)
        def _():
            base = cid * ECH
            pltpu.sync_copy(src.at[pl.ds(base, ECH)], six)
            pltpu.sync_copy(dst.at[pl.ds(base, ECH)], dix)
            cs = pltpu.async_copy(meta_s.at[six], rows_s, sem)
            cd = pltpu.async_copy(meta_d.at[dix], rows_d, sem)
            cs.wait()
            cd.wait()

            def row_add(e, _):
                rows_s[e, pl.ds(0, L)] = (rows_s[e, pl.ds(0, L)]
                                          + rows_d[e, pl.ds(0, L)])
                return 0
            lax.fori_loop(0, ECH, row_add, 0)
            pltpu.sync_copy(rows_s, comb_out.at[pl.ds(base, ECH), :])
        return 0
    lax.fori_loop(0, EIT, edge_it, 0)


@functools.partial(
    pl.kernel,
    out_type=jax.ShapeDtypeStruct((NC, N, D), F32),
    mesh=_MESH,
    scratch_types=[
        pltpu.VMEM((ECH,), jnp.int32),
        pltpu.VMEM((ECH,), jnp.int32),
        pltpu.VMEM((ECH,), jnp.int32),
        pltpu.VMEM((ECH, D), F32),
        pltpu.VMEM((ECH, D), F32),
        pltpu.VMEM((ECH, D), F32),
        pltpu.VMEM((8, D), F32),
        pltpu.VMEM_SHARED((ND, D), F32),
        pltpu.SemaphoreType.DMA,
    ],
)
def _sc_den2(qm, src, dixa, dixb, masks, zin, den_out,
             six, da, db, rows, rowsa, rowsb, mrow, den_sh, sem):
    c = lax.axis_index("c")
    s = lax.axis_index("s")
    wid = s * NC + c

    pltpu.sync_copy(masks, mrow)
    pltpu.sync_copy(zin.at[pl.ds(0, ECH), :], rowsa)
    pltpu.sync_copy(zin.at[pl.ds(0, ECH), :], rowsb)
    _zero_core_rows(den_sh, zin, s)
    plsc.subcore_barrier()

    def edge_it(it, _):
        cid = it * NW + wid

        @pl.when(cid < NECH)
        def _():
            base = cid * ECH
            pltpu.sync_copy(src.at[pl.ds(base, ECH)], six)
            pltpu.sync_copy(dixa.at[pl.ds(base, ECH)], da)
            pltpu.sync_copy(dixb.at[pl.ds(base, ECH)], db)
            pltpu.async_copy(qm.at[six], rows, sem).wait()
            m0 = mrow[0, pl.ds(0, L)]
            m1 = mrow[1, pl.ds(0, L)]

            def row_mask(e, _):
                r = rows[e, pl.ds(0, L)]
                rowsa[e, pl.ds(0, L)] = r * m0
                rowsb[e, pl.ds(0, L)] = r * m1
                return 0
            lax.fori_loop(0, ECH, row_mask, 0)
            pltpu.sync_copy(rowsa, den_sh.at[da], add=True)
            pltpu.sync_copy(rowsb, den_sh.at[db], add=True)
        return 0
    lax.fori_loop(0, EIT, edge_it, 0)
    plsc.subcore_barrier()
    _writeout_core_rows(den_sh, den_out, c, s)


R = 1000
G = N // R


def _row_spec(cols):
    return pl.BlockSpec((R, cols), lambda i: (i, 0))


def _deg_spec():
    return pl.BlockSpec((NC, R, D), lambda i: (0, i, 0))


def _full_spec(r, cols):
    return pl.BlockSpec((r, cols), lambda i: (0, 0))


def _indeg(dref):
    return dref[0, :, 0] + dref[1, :, 0]


def _tca_body(u, i, pu, pi, d, W1u, W1i, b1, W2, b2, Wg, hg_o, g_o):
    lane = lax.broadcasted_iota(jnp.int32, (R, 2 * EMB), 1)
    low = (lane < EMB).astype(F32)
    mu = low * (1.0 - pu[:]) + (1.0 - low) * pu[:]
    mi = low * (1.0 - pi[:]) + (1.0 - low) * pi[:]
    h1 = jnp.maximum(
        jnp.dot(u[:] * mu, W1u[:], preferred_element_type=F32)
        + jnp.dot(i[:] * mi, W1i[:], preferred_element_type=F32)
        + b1[:], 0.0)
    h2 = jnp.maximum(jnp.dot(h1, W2[:], preferred_element_type=F32) + b2[:], 0.0)
    hg = jnp.dot(h2, Wg[:], preferred_element_type=F32)
    dinv = lax.rsqrt(_indeg(d) + 1.0)
    hg_o[:] = hg
    g_o[:] = hg * dinv[:, None]


def _tc_mlp(u_pack, i_pack, pu, pi, deg, W1u, W1i, b1, W2, b2, Wg):
    return pl.pallas_call(
        _tca_body,
        grid=(G,),
        in_specs=[
            _row_spec(2 * EMB), _row_spec(2 * EMB),
            _row_spec(1), _row_spec(1), _deg_spec(),
            _full_spec(2 * EMB, 1024), _full_spec(2 * EMB, 1024),
            _full_spec(1, 1024),
            _full_spec(1024, 512), _full_spec(1, 512),
            _full_spec(512, D),
        ],
        out_specs=[_row_spec(D), _row_spec(D)],
        out_shape=[
            jax.ShapeDtypeStruct((N, D), F32),
            jax.ShapeDtypeStruct((N, D), F32),
        ],
    )(u_pack, i_pack, pu, pi, deg, W1u, W1i, b1, W2, b2, Wg)


def _tcb_body(a0, a1, hg, d, bg, gam, bet, mu, var, o_ref):
    dinv = lax.rsqrt(_indeg(d) + 1.0)
    o = (a0[:] + a1[:]) * dinv[:, None] + hg[:] * (dinv * dinv)[:, None] + bg[:]
    o = gam[:] * (o - mu[:]) / jnp.sqrt(var[:] + 1e-5) + bet[:]
    o_ref[:] = jnp.maximum(o, 0.0)


def _tc_gcn_bn(a0, a1, hg, deg, bg, gam, bet, mu, var):
    return pl.pallas_call(
        _tcb_body,
        grid=(G,),
        in_specs=[
            _row_spec(D), _row_spec(D), _row_spec(D), _deg_spec(),
            _full_spec(1, D), _full_spec(1, D), _full_spec(1, D),
            _full_spec(1, D), _full_spec(1, D),
        ],
        out_specs=[_row_spec(D)],
        out_shape=[jax.ShapeDtypeStruct((N, D), F32)],
    )(a0, a1, hg, deg, bg, gam, bet, mu, var)[0]


def _tcc_body(a0, a1, hbn, d, Wl, Wr, bs, h3_o, gc_o):
    indeg = _indeg(d)
    aggr = (a0[:] + a1[:]) / jnp.maximum(indeg, 1.0)[:, None]
    h3 = jnp.maximum(
        jnp.dot(aggr, Wl[:], preferred_element_type=F32)
        + jnp.dot(hbn[:], Wr[:], preferred_element_type=F32) + bs[:], 0.0)
    dinv_c = jnp.where(indeg > 0, lax.rsqrt(jnp.maximum(indeg, 1.0)), 0.0)
    h3_o[:] = h3
    gc_o[:] = h3 * dinv_c[:, None]


def _tc_sage(a0, a1, hbn, deg, Wl, Wr, bs):
    return pl.pallas_call(
        _tcc_body,
        grid=(G,),
        in_specs=[
            _row_spec(D), _row_spec(D), _row_spec(D), _deg_spec(),
            _full_spec(D, D), _full_spec(D, D), _full_spec(1, D),
        ],
        out_specs=[_row_spec(D), _row_spec(D)],
        out_shape=[
            jax.ShapeDtypeStruct((N, D), F32),
            jax.ShapeDtypeStruct((N, D), F32),
        ],
    )(a0, a1, hbn, deg, Wl, Wr, bs)


def _tcd_body(t0, t1, h3, d, W0, W1c, bc, Wgat, avs, avd,
              hg4_o, g1_o, g2_o, ms_o, md_o, qm_o):
    indeg = _indeg(d)
    dinv_c = jnp.where(indeg > 0, lax.rsqrt(jnp.maximum(indeg, 1.0)), 0.0)
    tx1 = -(t0[:] + t1[:]) * dinv_c[:, None]
    h4 = jnp.maximum(
        jnp.dot(h3[:], W0[:], preferred_element_type=F32)
        + jnp.dot(tx1, W1c[:], preferred_element_type=F32) + bc[:], 0.0)
    hg4 = jnp.dot(h4, Wgat[:], preferred_element_type=F32)
    asv = jnp.sum(hg4 * avs[:], axis=-1)
    adv = jnp.sum(hg4 * avd[:], axis=-1)
    q1 = jnp.exp(asv)
    q2 = jnp.exp(0.2 * asv)
    lane = lax.broadcasted_iota(jnp.int32, (R, D), 1)
    l0 = (lane == 0).astype(F32)
    l1 = (lane == 1).astype(F32)
    hg4_o[:] = hg4
    g1_o[:] = hg4 * q1[:, None]
    g2_o[:] = hg4 * q2[:, None]
    ms_o[:] = asv[:, None] * l0
    md_o[:] = adv[:, None] * l1
    qm_o[:] = q1[:, None] * l0 + q2[:, None] * l1


def _tc_cheb_gatprep(t0, t1, h3, deg, W0, W1c, bc, Wgat, avs, avd):
    return pl.pallas_call(
        _tcd_body,
        grid=(G,),
        in_specs=[
            _row_spec(D), _row_spec(D), _row_spec(D), _deg_spec(),
            _full_spec(D, D), _full_spec(D, D), _full_spec(1, D),
            _full_spec(D, D), _full_spec(1, D), _full_spec(1, D),
        ],
        out_specs=[_row_spec(D), _row_spec(D), _row_spec(D),
                   _row_spec(D), _row_spec(D), _row_spec(D)],
        out_shape=[jax.ShapeDtypeStruct((N, D), F32)] * 6,
    )(t0, t1, h3, deg, W0, W1c, bc, Wgat, avs, avd)


EB = 4000
EG = E // EB


def _edge_spec(cols):
    return pl.BlockSpec((EB, cols), lambda i: (i, 0))


def _tcf_body(comb, dstb, dixa_o, dixb_o):
    t = comb[:, 0:1] + comb[:, 1:2]
    keep = t >= 0.0
    d = dstb[:]
    dixa_o[:] = jnp.where(keep, d, N)
    dixb_o[:] = jnp.where(keep, N, d)


def _tc_edge(comb, dst2d):
    return pl.pallas_call(
        _tcf_body,
        grid=(EG,),
        in_specs=[_edge_spec(D), _edge_spec(1)],
        out_specs=[_edge_spec(1), _edge_spec(1)],
        out_shape=[
            jax.ShapeDtypeStruct((E, 1), jnp.int32),
            jax.ShapeDtypeStruct((E, 1), jnp.int32),
        ],
    )(comb, dst2d)


def _tce_body(na0, na1, nb0, nb1, d0, d1, hg4, ms, md,
              bgat, Wp, bp, o_ref):
    a = ms[:, 0]
    b = md[:, 1]
    qd1 = jnp.exp(b)
    qd2 = jnp.exp(0.2 * b)
    t = a + b
    wself = jnp.exp(jnp.maximum(t, 0.2 * t))
    num = (qd1[:, None] * (na0[:] + na1[:])
           + qd2[:, None] * (nb0[:] + nb1[:])
           + wself[:, None] * hg4[:])
    den = (qd1 * (d0[:, 0] + d1[:, 0])
           + qd2 * (d0[:, 1] + d1[:, 1]) + wself)
    o = num / jnp.maximum(den, 1e-16)[:, None]
    h5 = o + bgat[:]
    h5 = jnp.where(h5 > 0, h5, jnp.exp(jnp.minimum(h5, 0.0)) - 1.0)
    o_ref[:] = jnp.dot(h5, Wp[:], preferred_element_type=F32) + bp[:]


def _tc_gat_pred(na0, na1, nb0, nb1, d0, d1, hg4, ms, md, bgat, Wp, bp):
    return pl.pallas_call(
        _tce_body,
        grid=(G,),
        in_specs=[
            _row_spec(D), _row_spec(D), _row_spec(D), _row_spec(D),
            _row_spec(D), _row_spec(D),
            _row_spec(D), _row_spec(D), _row_spec(D),
            _full_spec(1, D), _full_spec(D, 1), _full_spec(1, 1),
        ],
        out_specs=[_row_spec(1)],
        out_shape=[jax.ShapeDtypeStruct((N, 1), F32)],
    )(na0, na1, nb0, nb1, d0, d1, hg4, ms, md, bgat, Wp, bp)[0]


def kernel(x, edge_index, batch, user_table, item_table, W1, b1, W2, b2,
           W_gcn, b_gcn, bn_gamma, bn_beta, bn_mean, bn_var, W_sage_l,
           W_sage_r, b_sage, W_cheb0, W_cheb1, b_cheb, W_gat, a_src, a_dst,
           b_gat, W_pred, b_pred):
    u_idx = x[:, 0].astype(jnp.int32)
    i_idx = x[:, 1].astype(jnp.int32)
    src = edge_index[0].astype(jnp.int32)
    dst = edge_index[1].astype(jnp.int32)
    user_p = user_table.reshape(-1, 2 * EMB)
    item_p = item_table.reshape(-1, 2 * EMB)
    pu = (u_idx & 1).astype(F32).reshape(N, 1)
    pi = (i_idx & 1).astype(F32).reshape(N, 1)
    W1u2 = jnp.concatenate([W1[0:EMB], W1[0:EMB]], axis=0)
    W1i2 = jnp.concatenate([W1[EMB:2 * EMB], W1[EMB:2 * EMB]], axis=0)
    ones_in = jnp.concatenate(
        [jnp.ones((ECH, 1), F32), jnp.zeros((ECH, D - 1), F32)], axis=1)
    zin = jnp.zeros((WRB, D), F32)
    b1r = b1.reshape(1, -1)
    b2r = b2.reshape(1, -1)
    bgr = b_gcn.reshape(1, -1)
    gam = bn_gamma.reshape(1, -1)
    bet = bn_beta.reshape(1, -1)
    mu = bn_mean.reshape(1, -1)
    var = bn_var.reshape(1, -1)
    bsr = b_sage.reshape(1, -1)
    bcr = b_cheb.reshape(1, -1)
    bga = b_gat.reshape(1, -1)

    u_pack, i_pack, deg = _sc_embed_deg(user_p, item_p, u_idx >> 1,
                                        i_idx >> 1, dst, ones_in, zin)
    hg, g = _tc_mlp(u_pack, i_pack, pu, pi, deg, W1u2, W1i2, b1r, W2, b2r,
                    W_gcn)

    acc = _sc_edge_sum(g, src, dst, zin)
    hbn = _tc_gcn_bn(acc[0], acc[1], hg, deg, bgr, gam, bet, mu, var)

    aggr = _sc_edge_sum(hbn, src, dst, zin)
    h3, gc = _tc_sage(aggr[0], aggr[1], hbn, deg, W_sage_l, W_sage_r, bsr)

    tx = _sc_edge_sum(gc, src, dst, zin)
    hg4, g1, g2, ms, md, qm = _tc_cheb_gatprep(
        tx[0], tx[1], h3, deg, W_cheb0, W_cheb1, bcr, W_gat, a_src, a_dst)

    comb = _sc_expand(ms, md, src, dst)
    dixa, dixb = _tc_edge(comb, dst.reshape(E, 1))
    dixa = dixa.reshape(-1)
    dixb = dixb.reshape(-1)
    numa = _sc_edge_sum(g1, src, dixa, zin)
    numa, g2, dixb2 = lax.optimization_barrier((numa, g2, dixb))
    numb = _sc_edge_sum(g2, src, dixb2, zin)
    numb, qm1, dixa2, dixb3 = lax.optimization_barrier((numb, qm, dixa, dixb))
    lane = jnp.arange(D)
    masks = jnp.stack([(lane == 0).astype(F32), (lane == 1).astype(F32)]
                      + [jnp.zeros((D,), F32)] * 6, axis=0)
    den = _sc_den2(qm1, src, dixa2, dixb3, masks, zin)

    out = _tc_gat_pred(numa[0], numa[1], numb[0], numb[1],
                       den[0], den[1],
                       hg4, ms, md, bga, W_pred, b_pred.reshape(1, 1))
    return out.reshape(-1)
